# Initial kernel scaffold; baseline (speedup 1.0000x reference)
import jax
import jax.numpy as jnp
from jax.experimental import pallas as pl


def kernel(x, edge_index, batch, vx, W0_w, W0_b, W1_w, W1_b, bn_gamma, bn_beta, bn_mean, bn_var):
    raise NotImplementedError("write your pallas kernel here")


# speedup vs baseline: 5.0725x; 5.0725x over previous
import jax
import jax.numpy as jnp
from jax import lax
from jax.experimental import pallas as pl
from jax.experimental.pallas import tpu as pltpu

N_NODES = 10000
D = 256
N_GRAPHS = 512
BLOCK = 400
GRID = N_NODES // BLOCK


def _fused_body(x_ref, batch_ref, vx_ref, W0_ref, W1_ref, bsum_ref, s_ref, t_ref,
                h_ref, v_ref, pooled_acc):
    i = pl.program_id(0)

    ids = batch_ref[0, 0, :]
    M = (ids[:, None] == lax.broadcasted_iota(jnp.int32, (BLOCK, N_GRAPHS), 1)
         ).astype(jnp.float32)

    g = jnp.dot(M, vx_ref[...], preferred_element_type=jnp.float32)
    h = x_ref[...] + g
    h_ref[...] = h

    part = lax.dot_general(M, h, (((0,), (0,)), ((), ())),
                           preferred_element_type=jnp.float32)

    @pl.when(i == 0)
    def _():
        pooled_acc[...] = part

    @pl.when(i > 0)
    def _():
        pooled_acc[...] += part

    @pl.when(i == GRID - 1)
    def _():
        A = lax.dot_general(vx_ref[...], W0_ref[...], (((1,), (1,)), ((), ())),
                            preferred_element_type=jnp.float32)
        P = lax.dot_general(pooled_acc[...], W1_ref[...], (((1,), (1,)), ((), ())),
                            preferred_element_type=jnp.float32)
        v = (A + P + bsum_ref[...]) * s_ref[...] + t_ref[...]
        v_ref[...] = jnp.maximum(v, 0.0)


def kernel(x, edge_index, batch, vx, W0_w, W0_b, W1_w, W1_b,
           bn_gamma, bn_beta, bn_mean, bn_var):
    del edge_index
    s = bn_gamma * lax.rsqrt(bn_var + 1e-5)
    t = bn_beta - bn_mean * s
    bsum = (W0_b + W1_b).reshape(1, D)
    s2 = s.reshape(1, D)
    t2 = t.reshape(1, D)
    batch3 = batch.reshape(GRID, 1, BLOCK)

    h, v = pl.pallas_call(
        _fused_body,
        grid=(GRID,),
        in_specs=[
            pl.BlockSpec((BLOCK, D), lambda i: (i, 0)),
            pl.BlockSpec((1, 1, BLOCK), lambda i: (i, 0, 0)),
            pl.BlockSpec((N_GRAPHS, D), lambda i: (0, 0)),
            pl.BlockSpec((D, D), lambda i: (0, 0)),
            pl.BlockSpec((D, D), lambda i: (0, 0)),
            pl.BlockSpec((1, D), lambda i: (0, 0)),
            pl.BlockSpec((1, D), lambda i: (0, 0)),
            pl.BlockSpec((1, D), lambda i: (0, 0)),
        ],
        out_specs=[
            pl.BlockSpec((BLOCK, D), lambda i: (i, 0)),
            pl.BlockSpec((N_GRAPHS, D), lambda i: (0, 0)),
        ],
        out_shape=[
            jax.ShapeDtypeStruct((N_NODES, D), jnp.float32),
            jax.ShapeDtypeStruct((N_GRAPHS, D), jnp.float32),
        ],
        scratch_shapes=[pltpu.VMEM((N_GRAPHS, D), jnp.float32)],
    )(x, batch3, vx, W0_w, W1_w, bsum, s2, t2)
    return (h, v)
